# baseline (speedup 1.0000x reference)
import jax
import jax.numpy as jnp
from jax import lax
from jax.experimental import pallas as pl
from jax.experimental.pallas import tpu as pltpu
from jax.experimental.pallas import tpu_sc as plsc

B, S, D = 4, 8192, 2048
NW = 32
DCHUNK = D // (NW // B)
NDC = D // DCHUNK
SB = 64
NSB = S // SB
NRING = 2
NV = DCHUNK // 16


def _cumsum_body(x_hbm, out_hbm, in_buf, out_buf, in_sems, out_sems):
    c = lax.axis_index("c")
    s = lax.axis_index("s")
    wid = c * 16 + s
    b = wid // NDC
    dc = (wid % NDC) * DCHUNK

    def gather(blk, slot):
        return pltpu.make_async_copy(
            x_hbm.at[b, pl.ds(blk * SB, SB), pl.ds(dc, DCHUNK)],
            in_buf.at[slot],
            in_sems.at[slot],
        )

    def scatter(blk, slot):
        return pltpu.make_async_copy(
            out_buf.at[slot],
            out_hbm.at[b, pl.ds(blk * SB, SB), pl.ds(dc, DCHUNK)],
            out_sems.at[slot],
        )

    for k in range(NRING):
        gather(k, k).start()

    def outer(g, accs):
        for k in range(NRING):
            blk = g * NRING + k
            gather(blk, k).wait()

            @pl.when(g > 0)
            def _():
                scatter(blk - NRING, k).wait()

            def step(r, accs):
                new = []
                for j in range(NV):
                    a = accs[j] + in_buf[k, r, pl.ds(j * 16, 16)]
                    out_buf[k, r, pl.ds(j * 16, 16)] = a
                    new.append(a)
                return tuple(new)

            accs = lax.fori_loop(0, SB, step, accs, unroll=2)
            scatter(blk, k).start()

            @pl.when(g < NSB // NRING - 1)
            def _():
                gather(blk + NRING, k).start()
        return accs

    zeros = tuple(jnp.zeros((16,), jnp.float32) for _ in range(NV))
    lax.fori_loop(0, NSB // NRING, outer, zeros)

    for k in range(NRING):
        scatter(NSB - NRING + k, k).wait()


@jax.jit
def kernel(x):
    run = pl.kernel(
        _cumsum_body,
        out_type=jax.ShapeDtypeStruct((B, S, D), jnp.float32),
        mesh=plsc.VectorSubcoreMesh(core_axis_name="c", subcore_axis_name="s"),
        scratch_types=[
            pltpu.VMEM((NRING, SB, DCHUNK), jnp.float32),
            pltpu.VMEM((NRING, SB, DCHUNK), jnp.float32),
            pltpu.SemaphoreType.DMA((NRING,)),
            pltpu.SemaphoreType.DMA((NRING,)),
        ],
    )
    return run(x)


# speedup vs baseline: 1.0020x; 1.0020x over previous
import jax
import jax.numpy as jnp
from jax import lax
from jax.experimental import pallas as pl
from jax.experimental.pallas import tpu as pltpu
from jax.experimental.pallas import tpu_sc as plsc

B, S, D = 4, 8192, 2048
NW = 32
DCHUNK = D // (NW // B)
NDC = D // DCHUNK
SB = 64
NSB = S // SB
NRING = 2
NV = DCHUNK // 16


def _cumsum_body(x_hbm, out_hbm, in_buf, out_buf, in_sems, out_sems):
    c = lax.axis_index("c")
    s = lax.axis_index("s")
    wid = s * 2 + c
    b = wid // NDC
    dc = (wid % NDC) * DCHUNK

    def gather(blk, slot):
        return pltpu.make_async_copy(
            x_hbm.at[b, pl.ds(blk * SB, SB), pl.ds(dc, DCHUNK)],
            in_buf.at[slot],
            in_sems.at[slot],
        )

    def scatter(blk, slot):
        return pltpu.make_async_copy(
            out_buf.at[slot],
            out_hbm.at[b, pl.ds(blk * SB, SB), pl.ds(dc, DCHUNK)],
            out_sems.at[slot],
        )

    for k in range(NRING):
        gather(k, k).start()

    def outer(g, accs):
        for k in range(NRING):
            blk = g * NRING + k
            gather(blk, k).wait()

            @pl.when(g > 0)
            def _():
                scatter(blk - NRING, k).wait()

            def step(r, accs):
                new = []
                for j in range(NV):
                    a = accs[j] + in_buf[k, r, pl.ds(j * 16, 16)]
                    out_buf[k, r, pl.ds(j * 16, 16)] = a
                    new.append(a)
                return tuple(new)

            accs = lax.fori_loop(0, SB, step, accs, unroll=2)
            scatter(blk, k).start()

            @pl.when(g < NSB // NRING - 1)
            def _():
                gather(blk + NRING, k).start()
        return accs

    zeros = tuple(jnp.zeros((16,), jnp.float32) for _ in range(NV))
    lax.fori_loop(0, NSB // NRING, outer, zeros)

    for k in range(NRING):
        scatter(NSB - NRING + k, k).wait()


@jax.jit
def kernel(x):
    run = pl.kernel(
        _cumsum_body,
        out_type=jax.ShapeDtypeStruct((B, S, D), jnp.float32),
        mesh=plsc.VectorSubcoreMesh(core_axis_name="c", subcore_axis_name="s"),
        scratch_types=[
            pltpu.VMEM((NRING, SB, DCHUNK), jnp.float32),
            pltpu.VMEM((NRING, SB, DCHUNK), jnp.float32),
            pltpu.SemaphoreType.DMA((NRING,)),
            pltpu.SemaphoreType.DMA((NRING,)),
        ],
    )
    return run(x)
